# baseline (device time: 107624 ns/iter reference)
import jax
import jax.numpy as jnp
from jax import lax
from jax.experimental import pallas as pl
from jax.experimental.pallas import tpu as pltpu

N_DEV = 4
SQ = 2048
D_MODEL = 1024
N_HEADS = 8
DH = 128
SCALE = 0.08838834764831843

RB = 512
HALF = SQ // 2
CH = HALF // N_DEV


def _qkv_body(x_ref, wq_ref, wk_ref, wv_ref, cos_ref, sin_ref,
              q_ref, k_ref, v_ref):
    lane = lax.broadcasted_iota(jnp.int32, (RB, D_MODEL), 1)
    is_even = (lane % 2) == 0
    cosv = jnp.tile(cos_ref[...], (1, N_HEADS))
    sinv = jnp.tile(sin_ref[...], (1, N_HEADS))

    def rope(t, scale):
        t_next = pltpu.roll(t, D_MODEL - 1, 1)
        t_prev = pltpu.roll(t, 1, 1)
        t_rot = jnp.where(is_even, -t_next, t_prev)
        return (t * cosv + t_rot * sinv) * scale

    xb = x_ref[0].astype(jnp.bfloat16)
    q_ref[...] = rope(jnp.dot(xb, wq_ref[...],
                              preferred_element_type=jnp.float32),
                      SCALE).astype(jnp.bfloat16)
    k_ref[...] = rope(jnp.dot(xb, wk_ref[...],
                              preferred_element_type=jnp.float32),
                      1.0).astype(jnp.bfloat16)
    v_ref[...] = jnp.dot(xb, wv_ref[...],
                         preferred_element_type=jnp.float32
                         ).astype(jnp.bfloat16)


def _fused_body(q_ref, k_ref, v_ref, wo_ref, out_ref,
                rss_p, rsr_p, ag_p, rss_m, rsr_m, ag_m,
                send_p, recv_p, send_m, recv_m):
    my_pos = lax.axis_index("i")
    left = (my_pos + N_DEV - 1) % N_DEV
    right = (my_pos + 1) % N_DEV

    barrier_sem = pltpu.get_barrier_semaphore()
    for nbr in (left, right):
        pl.semaphore_signal(
            barrier_sem, inc=1,
            device_id=(nbr,), device_id_type=pl.DeviceIdType.MESH,
        )
    pl.semaphore_wait(barrier_sem, 2)

    wob = wo_ref[...]

    def compute_chunk(cidx):
        rows = pl.ds(cidx * CH, CH)
        ctx_list = []
        for h in range(N_HEADS):
            hsl = pl.ds(h * DH, DH)
            s = lax.dot_general(
                q_ref[rows, hsl], k_ref[:, hsl], (((1,), (1,)), ((), ())),
                preferred_element_type=jnp.float32,
            )
            p = jnp.exp(s)
            recip = 1.0 / jnp.sum(p, axis=-1, keepdims=True)
            ctx_h = jnp.dot(p.astype(jnp.bfloat16), v_ref[:, hsl],
                            preferred_element_type=jnp.float32)
            ctx_list.append((ctx_h * recip).astype(jnp.bfloat16))
        ctx = jnp.concatenate(ctx_list, axis=1)
        return jnp.dot(ctx, wob, preferred_element_type=jnp.float32)

    def rows_p(idx):
        return pl.ds(idx * CH, CH)

    def rows_m(idx):
        return pl.ds(HALF + idx * CH, CH)

    def copy(src, dst, ssem, rsem, dev):
        return pltpu.make_async_remote_copy(
            src_ref=src, dst_ref=dst, send_sem=ssem, recv_sem=rsem,
            device_id=(dev,), device_id_type=pl.DeviceIdType.MESH,
        )

    rss_p[0] = compute_chunk(my_pos).astype(jnp.bfloat16)
    rss_m[0] = compute_chunk(N_DEV + my_pos).astype(jnp.bfloat16)
    rp = copy(rss_p.at[0], rsr_p.at[0], send_p.at[0], recv_p.at[0], right)
    rm = copy(rss_m.at[0], rsr_m.at[0], send_m.at[0], recv_m.at[0], left)
    rp.start()
    rm.start()
    for s in range(N_DEV - 1):
        ridx_p = (my_pos - s - 1 + N_DEV) % N_DEV
        ridx_m = (my_pos + s + 1) % N_DEV
        pd_top = compute_chunk(ridx_p)
        pd_bot = compute_chunk(N_DEV + ridx_m)
        rp.wait()
        rm.wait()
        acc_p = pd_top + rsr_p[s].astype(jnp.float32)
        acc_m = pd_bot + rsr_m[s].astype(jnp.float32)
        if s < N_DEV - 2:
            rss_p[s + 1] = acc_p.astype(jnp.bfloat16)
            rss_m[s + 1] = acc_m.astype(jnp.bfloat16)
            rp = copy(rss_p.at[s + 1], rsr_p.at[s + 1],
                      send_p.at[s + 1], recv_p.at[s + 1], right)
            rm = copy(rss_m.at[s + 1], rsr_m.at[s + 1],
                      send_m.at[s + 1], recv_m.at[s + 1], left)
        else:
            out_ref[0, rows_p(ridx_p), :] = acc_p
            out_ref[0, rows_m(ridx_m), :] = acc_m
            ag_p[0] = acc_p.astype(jnp.bfloat16)
            ag_m[0] = acc_m.astype(jnp.bfloat16)
            t = N_DEV - 1
            rp = copy(ag_p.at[0], ag_p.at[1],
                      send_p.at[t], recv_p.at[t], right)
            rm = copy(ag_m.at[0], ag_m.at[1],
                      send_m.at[t], recv_m.at[t], left)
        rp.start()
        rm.start()

    for s in range(N_DEV - 1):
        rp.wait()
        rm.wait()
        if s < N_DEV - 2:
            t = N_DEV + s
            rp = copy(ag_p.at[s + 1], ag_p.at[s + 2],
                      send_p.at[t], recv_p.at[t], right)
            rm = copy(ag_m.at[s + 1], ag_m.at[s + 2],
                      send_m.at[t], recv_m.at[t], left)
            rp.start()
            rm.start()
        gidx_p = (my_pos - s + N_DEV) % N_DEV
        gidx_m = (my_pos + s) % N_DEV
        out_ref[0, rows_p(gidx_p), :] = ag_p[s + 1].astype(jnp.float32)
        out_ref[0, rows_m(gidx_m), :] = ag_m[s + 1].astype(jnp.float32)


def kernel(x, Wq, Wk, Wv, Wo):
    wq = Wq.astype(jnp.bfloat16)
    wk = Wk.astype(jnp.bfloat16)
    wv = Wv.astype(jnp.bfloat16)
    wo = Wo.astype(jnp.bfloat16)

    import numpy as np
    pos = np.arange(SQ, dtype=np.float32)[:, None]
    freq_even = (np.arange(DH) // 2 * 2).astype(np.float32)
    inv = np.exp(freq_even * (-np.log(10000.0) / DH))[None, :]
    angle = pos * inv
    cos_t = jnp.asarray(np.cos(angle), dtype=jnp.float32)
    sin_t = jnp.asarray(np.sin(angle), dtype=jnp.float32)

    bf = jnp.bfloat16
    q, k, v = pl.pallas_call(
        _qkv_body,
        grid=(SQ // RB,),
        out_shape=[jax.ShapeDtypeStruct((SQ, D_MODEL), bf)] * 3,
        in_specs=[
            pl.BlockSpec((1, RB, D_MODEL), lambda r: (0, r, 0)),
            pl.BlockSpec((D_MODEL, D_MODEL), lambda r: (0, 0)),
            pl.BlockSpec((D_MODEL, D_MODEL), lambda r: (0, 0)),
            pl.BlockSpec((D_MODEL, D_MODEL), lambda r: (0, 0)),
            pl.BlockSpec((RB, DH), lambda r: (r, 0)),
            pl.BlockSpec((RB, DH), lambda r: (r, 0)),
        ],
        out_specs=[pl.BlockSpec((RB, D_MODEL), lambda r: (r, 0))] * 3,
        compiler_params=pltpu.CompilerParams(
            dimension_semantics=("arbitrary",),
        ),
    )(x, wq, wk, wv, cos_t, sin_t)

    nsteps = N_DEV - 1
    out = pl.pallas_call(
        _fused_body,
        out_shape=jax.ShapeDtypeStruct((1, SQ, D_MODEL), jnp.float32),
        in_specs=[pl.BlockSpec(memory_space=pltpu.VMEM)] * 4,
        out_specs=pl.BlockSpec(memory_space=pltpu.VMEM),
        scratch_shapes=[
            pltpu.VMEM((nsteps, CH, D_MODEL), bf),
            pltpu.VMEM((nsteps, CH, D_MODEL), bf),
            pltpu.VMEM((N_DEV, CH, D_MODEL), bf),
            pltpu.VMEM((nsteps, CH, D_MODEL), bf),
            pltpu.VMEM((nsteps, CH, D_MODEL), bf),
            pltpu.VMEM((N_DEV, CH, D_MODEL), bf),
            pltpu.SemaphoreType.DMA((2 * nsteps,)),
            pltpu.SemaphoreType.DMA((2 * nsteps,)),
            pltpu.SemaphoreType.DMA((2 * nsteps,)),
            pltpu.SemaphoreType.DMA((2 * nsteps,)),
        ],
        compiler_params=pltpu.CompilerParams(collective_id=0),
    )(q, k, v, wo)
    return out


# device time: 97619 ns/iter; 1.1025x vs baseline; 1.1025x over previous
import jax
import jax.numpy as jnp
from jax import lax
from jax.experimental import pallas as pl
from jax.experimental.pallas import tpu as pltpu

N_DEV = 4
SQ = 2048
D_MODEL = 1024
N_HEADS = 8
DH = 128
SCALE = 0.08838834764831843

RB = 512
HALF = SQ // 2
CH = HALF // N_DEV


def _qkv_body(x_ref, wq_ref, wk_ref, wv_ref, cos_ref, sin_ref,
              q_ref, k_ref, v_ref):
    lane = lax.broadcasted_iota(jnp.int32, (RB, D_MODEL), 1)
    is_even = (lane % 2) == 0
    cosv = jnp.tile(cos_ref[...], (1, N_HEADS))
    sinv = jnp.tile(sin_ref[...], (1, N_HEADS))

    def rope(t, scale):
        t_next = pltpu.roll(t, D_MODEL - 1, 1)
        t_prev = pltpu.roll(t, 1, 1)
        t_rot = jnp.where(is_even, -t_next, t_prev)
        return (t * cosv + t_rot * sinv) * scale

    xb = x_ref[0].astype(jnp.bfloat16)
    q_ref[...] = rope(jnp.dot(xb, wq_ref[...],
                              preferred_element_type=jnp.float32),
                      SCALE).astype(jnp.bfloat16)
    k_ref[...] = rope(jnp.dot(xb, wk_ref[...],
                              preferred_element_type=jnp.float32),
                      1.0).astype(jnp.bfloat16)
    v_ref[...] = jnp.dot(xb, wv_ref[...],
                         preferred_element_type=jnp.float32
                         ).astype(jnp.bfloat16)


def _fused_body(q_ref, k_ref, v_ref, wo_ref, out_ref,
                rss_p, rsr_p, ag_p, rss_m, rsr_m, ag_m,
                send_p, recv_p, send_m, recv_m):
    my_pos = lax.axis_index("i")
    left = (my_pos + N_DEV - 1) % N_DEV
    right = (my_pos + 1) % N_DEV

    barrier_sem = pltpu.get_barrier_semaphore()
    for nbr in (left, right):
        pl.semaphore_signal(
            barrier_sem, inc=1,
            device_id=(nbr,), device_id_type=pl.DeviceIdType.MESH,
        )
    pl.semaphore_wait(barrier_sem, 2)

    wob = wo_ref[...].astype(jnp.bfloat16)

    def compute_chunk(cidx):
        rows = pl.ds(cidx * CH, CH)
        ctx_list = []
        for h in range(N_HEADS):
            hsl = pl.ds(h * DH, DH)
            s = lax.dot_general(
                q_ref[rows, hsl], k_ref[:, hsl], (((1,), (1,)), ((), ())),
                preferred_element_type=jnp.float32,
            )
            p = jnp.exp(s)
            recip = 1.0 / jnp.sum(p, axis=-1, keepdims=True)
            ctx_h = jnp.dot(p.astype(jnp.bfloat16), v_ref[:, hsl],
                            preferred_element_type=jnp.float32)
            ctx_list.append((ctx_h * recip).astype(jnp.bfloat16))
        ctx = jnp.concatenate(ctx_list, axis=1)
        return jnp.dot(ctx, wob, preferred_element_type=jnp.float32)

    def rows_p(idx):
        return pl.ds(idx * CH, CH)

    def rows_m(idx):
        return pl.ds(HALF + idx * CH, CH)

    def copy(src, dst, ssem, rsem, dev):
        return pltpu.make_async_remote_copy(
            src_ref=src, dst_ref=dst, send_sem=ssem, recv_sem=rsem,
            device_id=(dev,), device_id_type=pl.DeviceIdType.MESH,
        )

    rss_p[0] = compute_chunk(my_pos).astype(jnp.bfloat16)
    rss_m[0] = compute_chunk(N_DEV + my_pos).astype(jnp.bfloat16)
    rp = copy(rss_p.at[0], rsr_p.at[0], send_p.at[0], recv_p.at[0], right)
    rm = copy(rss_m.at[0], rsr_m.at[0], send_m.at[0], recv_m.at[0], left)
    rp.start()
    rm.start()
    for s in range(N_DEV - 1):
        ridx_p = (my_pos - s - 1 + N_DEV) % N_DEV
        ridx_m = (my_pos + s + 1) % N_DEV
        pd_top = compute_chunk(ridx_p)
        pd_bot = compute_chunk(N_DEV + ridx_m)
        rp.wait()
        rm.wait()
        acc_p = pd_top + rsr_p[s].astype(jnp.float32)
        acc_m = pd_bot + rsr_m[s].astype(jnp.float32)
        if s < N_DEV - 2:
            rss_p[s + 1] = acc_p.astype(jnp.bfloat16)
            rss_m[s + 1] = acc_m.astype(jnp.bfloat16)
            rp = copy(rss_p.at[s + 1], rsr_p.at[s + 1],
                      send_p.at[s + 1], recv_p.at[s + 1], right)
            rm = copy(rss_m.at[s + 1], rsr_m.at[s + 1],
                      send_m.at[s + 1], recv_m.at[s + 1], left)
        else:
            ag_p[0] = acc_p.astype(jnp.bfloat16)
            ag_m[0] = acc_m.astype(jnp.bfloat16)
            out_ref[0, rows_p(ridx_p), :] = ag_p[0]
            out_ref[0, rows_m(ridx_m), :] = ag_m[0]
            t = N_DEV - 1
            rp = copy(ag_p.at[0], ag_p.at[1],
                      send_p.at[t], recv_p.at[t], right)
            rm = copy(ag_m.at[0], ag_m.at[1],
                      send_m.at[t], recv_m.at[t], left)
        rp.start()
        rm.start()

    for s in range(N_DEV - 1):
        rp.wait()
        rm.wait()
        if s < N_DEV - 2:
            t = N_DEV + s
            rp = copy(ag_p.at[s + 1], ag_p.at[s + 2],
                      send_p.at[t], recv_p.at[t], right)
            rm = copy(ag_m.at[s + 1], ag_m.at[s + 2],
                      send_m.at[t], recv_m.at[t], left)
            rp.start()
            rm.start()
        gidx_p = (my_pos - s + N_DEV) % N_DEV
        gidx_m = (my_pos + s) % N_DEV
        out_ref[0, rows_p(gidx_p), :] = ag_p[s + 1]
        out_ref[0, rows_m(gidx_m), :] = ag_m[s + 1]


def kernel(x, Wq, Wk, Wv, Wo):
    wq = Wq.astype(jnp.bfloat16)
    wk = Wk.astype(jnp.bfloat16)
    wv = Wv.astype(jnp.bfloat16)

    import numpy as np
    pos = np.arange(SQ, dtype=np.float32)[:, None]
    freq_even = (np.arange(DH) // 2 * 2).astype(np.float32)
    inv = np.exp(freq_even * (-np.log(10000.0) / DH))[None, :]
    angle = pos * inv
    cos_t = jnp.asarray(np.cos(angle), dtype=jnp.float32)
    sin_t = jnp.asarray(np.sin(angle), dtype=jnp.float32)

    bf = jnp.bfloat16
    q, k, v = pl.pallas_call(
        _qkv_body,
        grid=(SQ // RB,),
        out_shape=[jax.ShapeDtypeStruct((SQ, D_MODEL), bf)] * 3,
        in_specs=[
            pl.BlockSpec((1, RB, D_MODEL), lambda r: (0, r, 0)),
            pl.BlockSpec((D_MODEL, D_MODEL), lambda r: (0, 0)),
            pl.BlockSpec((D_MODEL, D_MODEL), lambda r: (0, 0)),
            pl.BlockSpec((D_MODEL, D_MODEL), lambda r: (0, 0)),
            pl.BlockSpec((RB, DH), lambda r: (r, 0)),
            pl.BlockSpec((RB, DH), lambda r: (r, 0)),
        ],
        out_specs=[pl.BlockSpec((RB, D_MODEL), lambda r: (r, 0))] * 3,
        compiler_params=pltpu.CompilerParams(
            dimension_semantics=("arbitrary",),
        ),
    )(x, wq, wk, wv, cos_t, sin_t)

    nsteps = N_DEV - 1
    out = pl.pallas_call(
        _fused_body,
        out_shape=jax.ShapeDtypeStruct((1, SQ, D_MODEL), jnp.bfloat16),
        in_specs=[pl.BlockSpec(memory_space=pltpu.VMEM)] * 4,
        out_specs=pl.BlockSpec(memory_space=pltpu.VMEM),
        scratch_shapes=[
            pltpu.VMEM((nsteps, CH, D_MODEL), bf),
            pltpu.VMEM((nsteps, CH, D_MODEL), bf),
            pltpu.VMEM((N_DEV, CH, D_MODEL), bf),
            pltpu.VMEM((nsteps, CH, D_MODEL), bf),
            pltpu.VMEM((nsteps, CH, D_MODEL), bf),
            pltpu.VMEM((N_DEV, CH, D_MODEL), bf),
            pltpu.SemaphoreType.DMA((2 * nsteps,)),
            pltpu.SemaphoreType.DMA((2 * nsteps,)),
            pltpu.SemaphoreType.DMA((2 * nsteps,)),
            pltpu.SemaphoreType.DMA((2 * nsteps,)),
        ],
        compiler_params=pltpu.CompilerParams(collective_id=0),
    )(q, k, v, Wo)
    return out
